# Initial kernel scaffold; baseline (speedup 1.0000x reference)
import jax
import jax.numpy as jnp
from jax.experimental import pallas as pl


def kernel(x, edge_attr, edge_index, nodes_idx, Wn, b_node, We, b_edge, fc_W, fce_W, attn_l, attn_r, attn_e, conv_b, gamma, beta, head_W, head_b):
    raise NotImplementedError("write your pallas kernel here")


# speedup vs baseline: 6.4801x; 6.4801x over previous
import functools

import jax
import jax.numpy as jnp
from jax import lax
from jax.experimental import pallas as pl
from jax.experimental.pallas import tpu as pltpu
from jax.experimental.pallas import tpu_sc as plsc

F32 = jnp.float32
I32 = jnp.int32

_NC = 2
_NS = 16
_LANES = 16
_CHUNK = 128


def _tca1_body(x_ref, wn_ref, bn_ref, alr_ref, nodes_ref, elr_ref, mel_ref):
    nodes = jnp.dot(x_ref[...], wn_ref[...], preferred_element_type=F32)
    nodes = nodes + bn_ref[...]
    nodes_ref[...] = nodes
    elr = jnp.dot(nodes, alr_ref[...], preferred_element_type=F32)
    elr_ref[...] = elr
    mel_ref[...] = jnp.max(elr, axis=0, keepdims=True)


def _tca2_body(ea_ref, aee_ref, bee_ref, ee_ref, mee_ref, *, n_real, blk):
    i = pl.program_id(0)
    ee = jnp.dot(ea_ref[...], aee_ref[...], preferred_element_type=F32)
    ee = ee + bee_ref[...]
    rows = i * blk + lax.broadcasted_iota(I32, ee.shape, 0)
    ee = jnp.where(rows < n_real, ee, -1e30)
    ee_ref[...] = ee
    m = jnp.max(ee, axis=0, keepdims=True)

    @pl.when(i == 0)
    def _():
        mee_ref[...] = m

    @pl.when(i > 0)
    def _():
        mee_ref[...] = jnp.maximum(mee_ref[...], m)


def _tcb_body(sn_ref, sx_ref, f1_ref, wef_ref, biase_ref, convb_ref,
              gamma_ref, beta_ref, hw_ref, hb_ref, hout_ref, pooled_ref,
              *, num_heads, edge_dim):
    hid = f1_ref.shape[1]
    outs = []
    for h in range(num_heads):
        sub = hid * (h % 2)
        sn_h = sn_ref[h // 2][:, sub:sub + hid]
        se_h = sx_ref[h // 2][:, sub:sub + edge_dim]
        esum = sx_ref[h // 2][:, sub + edge_dim:sub + edge_dim + 1]
        r = jnp.where(esum > 0, 1.0 / esum, 0.0)
        s = (esum > 0).astype(F32)
        rst = jnp.dot(sn_h, f1_ref[h], preferred_element_type=F32)
        rst = rst + jnp.dot(se_h, wef_ref[h], preferred_element_type=F32)
        rst = rst * r
        rst = rst + s * biase_ref[pl.ds(h, 1), :]
        rst = rst + convb_ref[pl.ds(h, 1), :]
        rst = rst * jax.nn.sigmoid(rst)
        outs.append(rst)
    n = outs[0].shape[0]
    cnt = float(n * num_heads)
    mu = sum(jnp.sum(o, axis=0, keepdims=True) for o in outs) / cnt
    var = sum(jnp.sum((o - mu) ** 2, axis=0, keepdims=True)
              for o in outs) / cnt
    scale = gamma_ref[...] / jnp.sqrt(var + 1e-5)
    normed = [(o - mu) * scale + beta_ref[...] for o in outs]
    y = jnp.concatenate(normed, axis=1)
    hout = jnp.dot(y, hw_ref[...], preferred_element_type=F32) + hb_ref[...]
    hout_ref[...] = hout
    pooled_ref[...] = jnp.sum(hout, axis=0, keepdims=True) / float(n)


def _sc_main_body(nodes_hbm, eler_hbm, ee_hbm, ea_hbm, src_hbm, dst_hbm,
                  m_hbm, z128_hbm,
                  sn_hbm, sx_hbm,
                  mv0, mv1, srcb, dstb, eeb, eab, rows, scl, elrows, errows,
                  acc, sem,
                  *, n_pad, e_pad, num_heads, node_dim, edge_dim):
    cid = lax.axis_index("c")
    sid = lax.axis_index("s")
    epw = e_pad // _NS
    nchunk = epw // _CHUNK
    npt = n_pad // _NS

    pltpu.sync_copy(m_hbm.at[cid * 2], mv0)
    pltpu.sync_copy(m_hbm.at[cid * 2 + 1], mv1)

    iot = lax.iota(I32, _LANES)
    ngroups = _CHUNK // _LANES
    rsl = pl.ds(sid * npt, npt)

    def zero_acc():
        pltpu.sync_copy(z128_hbm, scl)
        for q in range(npt // _CHUNK):
            qsl = pl.ds(sid * npt + q * _CHUNK, _CHUNK)
            pltpu.sync_copy(scl, acc.at[qsl])

    def eexp(hl, g):
        h16 = jnp.full((_LANES,), cid * 2 + hl, I32)
        mvec = (mv0, mv1)[hl][...]
        e0 = g * _LANES
        eloc = e0 + iot
        elg = plsc.load_gather(elrows, [eloc, h16])
        erg = plsc.load_gather(errows, [eloc, h16 + num_heads])
        eeg = plsc.load_gather(eeb, [eloc, h16])
        z = elg + erg + eeg
        zl = jnp.where(z > 0.0, z, 0.2 * z)
        cg = jnp.maximum(erg + mvec, 0.0)
        return eloc, jnp.exp(zl - cg)

    zero_acc()
    plsc.subcore_barrier()

    def chunk_a(k, carry):
        base = sid * epw + k * _CHUNK
        pltpu.sync_copy(src_hbm.at[pl.ds(base, _CHUNK)], srcb)
        pltpu.sync_copy(dst_hbm.at[pl.ds(base, _CHUNK)], dstb)
        pltpu.sync_copy(ee_hbm.at[pl.ds(base, _CHUNK), :], eeb)
        pltpu.async_copy(eler_hbm.at[srcb], elrows, sem).wait()
        pltpu.async_copy(eler_hbm.at[dstb], errows, sem).wait()
        pltpu.async_copy(nodes_hbm.at[srcb], rows, sem).wait()
        for hl in range(2):
            cbase = node_dim * hl

            def grp_a(g, carry2):
                eloc, exv = eexp(hl, g)

                def nd_body(d8, carry3):
                    for j in range(8):
                        dj16 = jnp.full((_LANES,), 0, I32) + (d8 * 8 + j)
                        v = plsc.load_gather(rows, [eloc, dj16])
                        plsc.store_scatter(scl, [eloc, dj16 + cbase], v * exv)
                    return carry3

                lax.fori_loop(0, node_dim // 8, nd_body, 0)
                return carry2

            lax.fori_loop(0, ngroups, grp_a, 0)
        pltpu.sync_copy(scl, acc.at[dstb], add=True)
        return carry

    lax.fori_loop(0, nchunk, chunk_a, 0)
    plsc.subcore_barrier()
    pltpu.sync_copy(acc.at[rsl], sn_hbm.at[cid, rsl, :])
    plsc.subcore_barrier()

    zero_acc()
    plsc.subcore_barrier()

    def chunk_b(k, carry):
        base = sid * epw + k * _CHUNK
        pltpu.sync_copy(src_hbm.at[pl.ds(base, _CHUNK)], srcb)
        pltpu.sync_copy(dst_hbm.at[pl.ds(base, _CHUNK)], dstb)
        pltpu.sync_copy(ee_hbm.at[pl.ds(base, _CHUNK), :], eeb)
        pltpu.sync_copy(ea_hbm.at[pl.ds(base, _CHUNK), :], eab)
        pltpu.async_copy(eler_hbm.at[srcb], elrows, sem).wait()
        pltpu.async_copy(eler_hbm.at[dstb], errows, sem).wait()
        for hl in range(2):
            cbase = node_dim * hl

            def grp_b(g, carry2):
                eloc, exv = eexp(hl, g)
                plsc.store_scatter(
                    scl, [eloc, jnp.full((_LANES,), cbase + edge_dim, I32)],
                    exv)
                for j in range(edge_dim):
                    j16 = jnp.full((_LANES,), j, I32)
                    v = plsc.load_gather(eab, [eloc, j16])
                    plsc.store_scatter(scl, [eloc, j16 + cbase], v * exv)
                return carry2

            lax.fori_loop(0, ngroups, grp_b, 0)
        pltpu.sync_copy(scl, acc.at[dstb], add=True)
        return carry

    lax.fori_loop(0, nchunk, chunk_b, 0)
    plsc.subcore_barrier()
    pltpu.sync_copy(acc.at[rsl], sx_hbm.at[cid, rsl, :])


def _sc_gather_body(hout_hbm, idx_hbm, out_hbm, idxv, rowsv, sem, *, b_per_w):
    wid = lax.axis_index("s") * _NC + lax.axis_index("c")
    base = wid * b_per_w
    pltpu.sync_copy(idx_hbm.at[pl.ds(base, b_per_w)], idxv)
    pltpu.async_copy(hout_hbm.at[idxv], rowsv, sem).wait()
    pltpu.sync_copy(rowsv, out_hbm.at[pl.ds(base, b_per_w)])


def kernel(x, edge_attr, edge_index, nodes_idx, Wn, b_node, We, b_edge,
           fc_W, fce_W, attn_l, attn_r, attn_e, conv_b, gamma, beta,
           head_W, head_b):
    n_nodes, node_dim_in = x.shape
    n_edges, edge_dim = edge_attr.shape
    hid = Wn.shape[1]
    num_heads = attn_l.shape[1]
    n_idx = nodes_idx.shape[0]

    fc1r = fc_W[-1].reshape(hid, num_heads, hid)
    fce1r = fce_W[-1].reshape(hid, num_heads, hid)
    al = jnp.einsum("khd,hd->kh", fc1r, attn_l[-1])
    ar = jnp.einsum("khd,hd->kh", fc1r, attn_r[-1])
    ae = jnp.einsum("khd,hd->kh", fce1r, attn_e[-1])
    alr = jnp.concatenate([al, ar], axis=1)
    aee = We @ ae
    bee = (b_edge @ ae).reshape(1, num_heads)
    f1 = jnp.transpose(fc1r, (1, 0, 2))
    fce1h = jnp.transpose(fce1r, (1, 0, 2))
    wef = jnp.einsum("ke,hed->hkd", We, fce1h)
    bias_e = jnp.einsum("e,hed->hd", b_edge, fce1h)

    n_pad = -(-n_nodes // (_NS * _CHUNK)) * (_NS * _CHUNK)
    e_pad = -(-n_edges // (_NS * _CHUNK)) * (_NS * _CHUNK)
    pad = e_pad - n_edges
    src = jnp.concatenate([edge_index[0], jnp.zeros((pad,), I32)])
    dst = jnp.concatenate([edge_index[1], jnp.zeros((pad,), I32)])
    eap = jnp.concatenate([edge_attr, jnp.zeros((pad, edge_dim), F32)], axis=0)

    nodes, elr, melr = pl.pallas_call(
        _tca1_body,
        out_shape=[
            jax.ShapeDtypeStruct((n_nodes, hid), F32),
            jax.ShapeDtypeStruct((n_nodes, 2 * num_heads), F32),
            jax.ShapeDtypeStruct((1, 2 * num_heads), F32),
        ],
    )(x, Wn, b_node.reshape(1, hid), alr)
    zrow = jnp.zeros((n_pad - n_nodes, num_heads), F32)
    el = jnp.concatenate([elr[:, :num_heads], zrow], axis=0)
    er = jnp.concatenate([elr[:, num_heads:], zrow], axis=0)

    gblk = e_pad // _NS
    ee, mee = pl.pallas_call(
        functools.partial(_tca2_body, n_real=n_edges, blk=gblk),
        grid=(_NS,),
        in_specs=[
            pl.BlockSpec((gblk, edge_dim), lambda i: (i, 0)),
            pl.BlockSpec((edge_dim, num_heads), lambda i: (0, 0)),
            pl.BlockSpec((1, num_heads), lambda i: (0, 0)),
        ],
        out_specs=[
            pl.BlockSpec((gblk, num_heads), lambda i: (i, 0)),
            pl.BlockSpec((1, num_heads), lambda i: (0, 0)),
        ],
        out_shape=[
            jax.ShapeDtypeStruct((e_pad, num_heads), F32),
            jax.ShapeDtypeStruct((1, num_heads), F32),
        ],
    )(eap, aee, bee)

    mstab = jnp.tile((melr[0, :num_heads] + mee[0]).reshape(num_heads, 1),
                     (1, _LANES))
    z128 = jnp.zeros((_CHUNK, 2 * hid), F32)

    eler = jnp.concatenate(
        [el, er, jnp.zeros((n_pad, _LANES - 2 * num_heads), F32)],
        axis=1)
    mesh = plsc.VectorSubcoreMesh(core_axis_name="c", subcore_axis_name="s")
    sc_main = functools.partial(
        pl.kernel,
        functools.partial(_sc_main_body, n_pad=n_pad, e_pad=e_pad,
                          num_heads=num_heads, node_dim=hid,
                          edge_dim=edge_dim),
        out_type=[
            jax.ShapeDtypeStruct((_NC, n_pad, 2 * hid), F32),
            jax.ShapeDtypeStruct((_NC, n_pad, 2 * hid), F32),
        ],
        mesh=mesh,
        scratch_types=[
            pltpu.VMEM((_LANES,), F32),
            pltpu.VMEM((_LANES,), F32),
            pltpu.VMEM((_CHUNK,), I32),
            pltpu.VMEM((_CHUNK,), I32),
            pltpu.VMEM((_CHUNK, num_heads), F32),
            pltpu.VMEM((_CHUNK, edge_dim), F32),
            pltpu.VMEM((_CHUNK, hid), F32),
            pltpu.VMEM((_CHUNK, 2 * hid), F32),
            pltpu.VMEM((_CHUNK, _LANES), F32),
            pltpu.VMEM((_CHUNK, _LANES), F32),
            pltpu.VMEM_SHARED((n_pad, 2 * hid), F32),
            pltpu.SemaphoreType.DMA,
        ],
        compiler_params=pltpu.CompilerParams(needs_layout_passes=False, use_tc_tiling_on_sc=False),
    )()
    sn, sx = sc_main(nodes, eler, ee, eap, src, dst, mstab, z128)
    sn = sn[:, :n_nodes, :]
    sx = sx[:, :n_nodes, :]

    hout, pooled = pl.pallas_call(
        functools.partial(_tcb_body, num_heads=num_heads, edge_dim=edge_dim),
        out_shape=[
            jax.ShapeDtypeStruct((n_nodes, hid), F32),
            jax.ShapeDtypeStruct((1, hid), F32),
        ],
        compiler_params=pltpu.CompilerParams(
            vmem_limit_bytes=100 * 1024 * 1024),
    )(sn, sx, f1, wef, bias_e, conv_b[-1].reshape(num_heads, hid),
      gamma[-1].reshape(1, hid), beta[-1].reshape(1, hid), head_W,
      head_b.reshape(1, hid))

    b_per_w = n_idx // (_NC * _NS)
    sc_gather = functools.partial(
        pl.kernel,
        functools.partial(_sc_gather_body, b_per_w=b_per_w),
        out_type=jax.ShapeDtypeStruct((n_idx, hid), F32),
        mesh=mesh,
        scratch_types=[
            pltpu.VMEM((b_per_w,), I32),
            pltpu.VMEM((b_per_w, hid), F32),
            pltpu.SemaphoreType.DMA,
        ],
        compiler_params=pltpu.CompilerParams(needs_layout_passes=False, use_tc_tiling_on_sc=False),
    )()
    gathered = sc_gather(hout, nodes_idx)

    return pooled, gathered


# speedup vs baseline: 6.6349x; 1.0239x over previous
import functools

import jax
import jax.numpy as jnp
from jax import lax
from jax.experimental import pallas as pl
from jax.experimental.pallas import tpu as pltpu
from jax.experimental.pallas import tpu_sc as plsc

F32 = jnp.float32
I32 = jnp.int32

_NC = 2
_NS = 16
_LANES = 16
_CHUNK = 128


def _tca1_body(x_ref, wn_ref, bn_ref, alr_ref, nodes_ref, elr_ref, mel_ref):
    nodes = jnp.dot(x_ref[...], wn_ref[...], preferred_element_type=F32)
    nodes = nodes + bn_ref[...]
    nodes_ref[...] = nodes
    elr = jnp.dot(nodes, alr_ref[...], preferred_element_type=F32)
    elr_ref[...] = elr
    mel_ref[...] = jnp.max(elr, axis=0, keepdims=True)


def _tca2_body(ea_ref, aee_ref, bee_ref, ee_ref, mee_ref, *, n_real, blk):
    i = pl.program_id(0)
    ee = jnp.dot(ea_ref[...], aee_ref[...], preferred_element_type=F32)
    ee = ee + bee_ref[...]
    rows = i * blk + lax.broadcasted_iota(I32, ee.shape, 0)
    ee = jnp.where(rows < n_real, ee, -1e30)
    ee_ref[...] = ee
    m = jnp.max(ee, axis=0, keepdims=True)

    @pl.when(i == 0)
    def _():
        mee_ref[...] = m

    @pl.when(i > 0)
    def _():
        mee_ref[...] = jnp.maximum(mee_ref[...], m)


def _tcb_body(sn_ref, sx_ref, f1_ref, wef_ref, biase_ref, convb_ref,
              gamma_ref, beta_ref, hw_ref, hb_ref, hout_ref, pooled_ref,
              *, num_heads, edge_dim):
    hid = f1_ref.shape[1]
    outs = []
    for h in range(num_heads):
        sub = hid * (h % 2)
        sn_h = sn_ref[h // 2][:, sub:sub + hid]
        se_h = sx_ref[h // 2][:, sub:sub + edge_dim]
        esum = sx_ref[h // 2][:, sub + edge_dim:sub + edge_dim + 1]
        r = jnp.where(esum > 0, 1.0 / esum, 0.0)
        s = (esum > 0).astype(F32)
        rst = jnp.dot(sn_h, f1_ref[h], preferred_element_type=F32)
        rst = rst + jnp.dot(se_h, wef_ref[h], preferred_element_type=F32)
        rst = rst * r
        rst = rst + s * biase_ref[pl.ds(h, 1), :]
        rst = rst + convb_ref[pl.ds(h, 1), :]
        rst = rst * jax.nn.sigmoid(rst)
        outs.append(rst)
    n = outs[0].shape[0]
    cnt = float(n * num_heads)
    mu = sum(jnp.sum(o, axis=0, keepdims=True) for o in outs) / cnt
    var = sum(jnp.sum((o - mu) ** 2, axis=0, keepdims=True)
              for o in outs) / cnt
    scale = gamma_ref[...] / jnp.sqrt(var + 1e-5)
    normed = [(o - mu) * scale + beta_ref[...] for o in outs]
    y = jnp.concatenate(normed, axis=1)
    hout = jnp.dot(y, hw_ref[...], preferred_element_type=F32) + hb_ref[...]
    hout_ref[...] = hout
    pooled_ref[...] = jnp.sum(hout, axis=0, keepdims=True) / float(n)


def _sc_main_body(nodes_hbm, eler_hbm, pk_hbm, m_hbm, z128_hbm,
                  sn_hbm, sx_hbm,
                  mv0, mv1, srcb, dstb, pkv, rows, scl, elrows, errows,
                  acc, sem,
                  *, n_pad, e_pad, num_heads, node_dim, edge_dim):
    cid = lax.axis_index("c")
    sid = lax.axis_index("s")
    epw = e_pad // _NS
    nchunk = epw // _CHUNK
    npt = n_pad // _NS
    pkw = 2 + num_heads + edge_dim + 2

    pltpu.sync_copy(m_hbm.at[cid * 2], mv0)
    pltpu.sync_copy(m_hbm.at[cid * 2 + 1], mv1)

    iot = lax.iota(I32, _LANES)
    ngroups = _CHUNK // _LANES
    rsl = pl.ds(sid * npt, npt)

    def zero_acc():
        pltpu.sync_copy(z128_hbm, scl)
        for q in range(npt // _CHUNK):
            qsl = pl.ds(sid * npt + q * _CHUNK, _CHUNK)
            pltpu.sync_copy(scl, acc.at[qsl])

    def load_chunk(k, with_rows):
        base = sid * epw + k * _CHUNK
        pltpu.sync_copy(pk_hbm.at[pl.ds(base, _CHUNK), :], pkv)
        for g in range(ngroups):
            eloc = g * _LANES + iot
            sv = plsc.load_gather(pkv, [eloc, jnp.zeros((_LANES,), I32)])
            dv = plsc.load_gather(pkv, [eloc, jnp.full((_LANES,), 1, I32)])
            srcb[pl.ds(g * _LANES, _LANES)] = plsc.bitcast(sv, I32)
            dstb[pl.ds(g * _LANES, _LANES)] = plsc.bitcast(dv, I32)
        cps = [pltpu.async_copy(eler_hbm.at[srcb], elrows, sem),
               pltpu.async_copy(eler_hbm.at[dstb], errows, sem)]
        if with_rows:
            cps.append(pltpu.async_copy(nodes_hbm.at[srcb], rows, sem))
        for cp in cps:
            cp.wait()

    def eexp(hl, g):
        h16 = jnp.full((_LANES,), cid * 2 + hl, I32)
        mvec = (mv0, mv1)[hl][...]
        eloc = g * _LANES + iot
        elg = plsc.load_gather(elrows, [eloc, h16])
        erg = plsc.load_gather(errows, [eloc, h16 + num_heads])
        eeg = plsc.load_gather(pkv, [eloc, h16 + 2])
        z = elg + erg + eeg
        zl = jnp.where(z > 0.0, z, 0.2 * z)
        cg = jnp.maximum(erg + mvec, 0.0)
        return eloc, jnp.exp(zl - cg)

    zero_acc()
    plsc.subcore_barrier()

    def chunk_a(k, carry):
        load_chunk(k, True)
        for hl in range(2):
            cbase = node_dim * hl

            def grp_a(g, carry2):
                eloc, exv = eexp(hl, g)
                for j in range(node_dim):
                    dj16 = jnp.full((_LANES,), j, I32)
                    v = plsc.load_gather(rows, [eloc, dj16])
                    plsc.store_scatter(scl, [eloc, dj16 + cbase], v * exv)
                return carry2

            lax.fori_loop(0, ngroups, grp_a, 0)
        pltpu.sync_copy(scl, acc.at[dstb], add=True)
        return carry

    lax.fori_loop(0, nchunk, chunk_a, 0)
    plsc.subcore_barrier()
    pltpu.sync_copy(acc.at[rsl], sn_hbm.at[cid, rsl, :])
    plsc.subcore_barrier()

    zero_acc()
    plsc.subcore_barrier()

    def chunk_b(k, carry):
        load_chunk(k, False)
        for hl in range(2):
            cbase = node_dim * hl

            def grp_b(g, carry2):
                eloc, exv = eexp(hl, g)
                plsc.store_scatter(
                    scl, [eloc, jnp.full((_LANES,), cbase + edge_dim, I32)],
                    exv)
                for j in range(edge_dim):
                    j16 = jnp.full((_LANES,), j + 2 + num_heads, I32)
                    v = plsc.load_gather(pkv, [eloc, j16])
                    plsc.store_scatter(
                        scl, [eloc, jnp.full((_LANES,), j + cbase, I32)],
                        v * exv)
                return carry2

            lax.fori_loop(0, ngroups, grp_b, 0)
        pltpu.sync_copy(scl, acc.at[dstb], add=True)
        return carry

    lax.fori_loop(0, nchunk, chunk_b, 0)
    plsc.subcore_barrier()
    pltpu.sync_copy(acc.at[rsl], sx_hbm.at[cid, rsl, :])


def _sc_gather_body(hout_hbm, idx_hbm, out_hbm, idxv, rowsv, sem, *, b_per_w):
    wid = lax.axis_index("s") * _NC + lax.axis_index("c")
    base = wid * b_per_w
    pltpu.sync_copy(idx_hbm.at[pl.ds(base, b_per_w)], idxv)
    pltpu.async_copy(hout_hbm.at[idxv], rowsv, sem).wait()
    pltpu.sync_copy(rowsv, out_hbm.at[pl.ds(base, b_per_w)])


def kernel(x, edge_attr, edge_index, nodes_idx, Wn, b_node, We, b_edge,
           fc_W, fce_W, attn_l, attn_r, attn_e, conv_b, gamma, beta,
           head_W, head_b):
    n_nodes, node_dim_in = x.shape
    n_edges, edge_dim = edge_attr.shape
    hid = Wn.shape[1]
    num_heads = attn_l.shape[1]
    n_idx = nodes_idx.shape[0]

    fc1r = fc_W[-1].reshape(hid, num_heads, hid)
    fce1r = fce_W[-1].reshape(hid, num_heads, hid)
    al = jnp.einsum("khd,hd->kh", fc1r, attn_l[-1])
    ar = jnp.einsum("khd,hd->kh", fc1r, attn_r[-1])
    ae = jnp.einsum("khd,hd->kh", fce1r, attn_e[-1])
    alr = jnp.concatenate([al, ar], axis=1)
    aee = We @ ae
    bee = (b_edge @ ae).reshape(1, num_heads)
    f1 = jnp.transpose(fc1r, (1, 0, 2))
    fce1h = jnp.transpose(fce1r, (1, 0, 2))
    wef = jnp.einsum("ke,hed->hkd", We, fce1h)
    bias_e = jnp.einsum("e,hed->hd", b_edge, fce1h)

    n_pad = -(-n_nodes // (_NS * _CHUNK)) * (_NS * _CHUNK)
    e_pad = -(-n_edges // (_NS * _CHUNK)) * (_NS * _CHUNK)
    pad = e_pad - n_edges
    src = jnp.concatenate([edge_index[0], jnp.zeros((pad,), I32)])
    dst = jnp.concatenate([edge_index[1], jnp.zeros((pad,), I32)])
    eap = jnp.concatenate([edge_attr, jnp.zeros((pad, edge_dim), F32)], axis=0)

    nodes, elr, melr = pl.pallas_call(
        _tca1_body,
        out_shape=[
            jax.ShapeDtypeStruct((n_nodes, hid), F32),
            jax.ShapeDtypeStruct((n_nodes, 2 * num_heads), F32),
            jax.ShapeDtypeStruct((1, 2 * num_heads), F32),
        ],
    )(x, Wn, b_node.reshape(1, hid), alr)
    zrow = jnp.zeros((n_pad - n_nodes, num_heads), F32)
    el = jnp.concatenate([elr[:, :num_heads], zrow], axis=0)
    er = jnp.concatenate([elr[:, num_heads:], zrow], axis=0)

    gblk = e_pad // _NS
    ee, mee = pl.pallas_call(
        functools.partial(_tca2_body, n_real=n_edges, blk=gblk),
        grid=(_NS,),
        in_specs=[
            pl.BlockSpec((gblk, edge_dim), lambda i: (i, 0)),
            pl.BlockSpec((edge_dim, num_heads), lambda i: (0, 0)),
            pl.BlockSpec((1, num_heads), lambda i: (0, 0)),
        ],
        out_specs=[
            pl.BlockSpec((gblk, num_heads), lambda i: (i, 0)),
            pl.BlockSpec((1, num_heads), lambda i: (0, 0)),
        ],
        out_shape=[
            jax.ShapeDtypeStruct((e_pad, num_heads), F32),
            jax.ShapeDtypeStruct((1, num_heads), F32),
        ],
    )(eap, aee, bee)

    mstab = jnp.tile((melr[0, :num_heads] + mee[0]).reshape(num_heads, 1),
                     (1, _LANES))
    z128 = jnp.zeros((_CHUNK, 2 * hid), F32)

    eler = jnp.concatenate(
        [el, er, jnp.zeros((n_pad, _LANES - 2 * num_heads), F32)],
        axis=1)
    pkw = 2 + num_heads + edge_dim + 2
    pk = jnp.concatenate([
        lax.bitcast_convert_type(src, F32).reshape(e_pad, 1),
        lax.bitcast_convert_type(dst, F32).reshape(e_pad, 1),
        ee,
        eap,
        jnp.zeros((e_pad, 2), F32),
    ], axis=1)
    mesh = plsc.VectorSubcoreMesh(core_axis_name="c", subcore_axis_name="s")
    sc_main = functools.partial(
        pl.kernel,
        functools.partial(_sc_main_body, n_pad=n_pad, e_pad=e_pad,
                          num_heads=num_heads, node_dim=hid,
                          edge_dim=edge_dim),
        out_type=[
            jax.ShapeDtypeStruct((_NC, n_pad, 2 * hid), F32),
            jax.ShapeDtypeStruct((_NC, n_pad, 2 * hid), F32),
        ],
        mesh=mesh,
        scratch_types=[
            pltpu.VMEM((_LANES,), F32),
            pltpu.VMEM((_LANES,), F32),
            pltpu.VMEM((_CHUNK,), I32),
            pltpu.VMEM((_CHUNK,), I32),
            pltpu.VMEM((_CHUNK, pkw), F32),
            pltpu.VMEM((_CHUNK, hid), F32),
            pltpu.VMEM((_CHUNK, 2 * hid), F32),
            pltpu.VMEM((_CHUNK, _LANES), F32),
            pltpu.VMEM((_CHUNK, _LANES), F32),
            pltpu.VMEM_SHARED((n_pad, 2 * hid), F32),
            pltpu.SemaphoreType.DMA,
        ],
        compiler_params=pltpu.CompilerParams(needs_layout_passes=False, use_tc_tiling_on_sc=False),
    )()
    sn, sx = sc_main(nodes, eler, pk, mstab, z128)
    sn = sn[:, :n_nodes, :]
    sx = sx[:, :n_nodes, :]

    hout, pooled = pl.pallas_call(
        functools.partial(_tcb_body, num_heads=num_heads, edge_dim=edge_dim),
        out_shape=[
            jax.ShapeDtypeStruct((n_nodes, hid), F32),
            jax.ShapeDtypeStruct((1, hid), F32),
        ],
        compiler_params=pltpu.CompilerParams(
            vmem_limit_bytes=100 * 1024 * 1024),
    )(sn, sx, f1, wef, bias_e, conv_b[-1].reshape(num_heads, hid),
      gamma[-1].reshape(1, hid), beta[-1].reshape(1, hid), head_W,
      head_b.reshape(1, hid))

    b_per_w = n_idx // (_NC * _NS)
    sc_gather = functools.partial(
        pl.kernel,
        functools.partial(_sc_gather_body, b_per_w=b_per_w),
        out_type=jax.ShapeDtypeStruct((n_idx, hid), F32),
        mesh=mesh,
        scratch_types=[
            pltpu.VMEM((b_per_w,), I32),
            pltpu.VMEM((b_per_w, hid), F32),
            pltpu.SemaphoreType.DMA,
        ],
        compiler_params=pltpu.CompilerParams(needs_layout_passes=False, use_tc_tiling_on_sc=False),
    )()
    gathered = sc_gather(hout, nodes_idx)

    return pooled, gathered


# speedup vs baseline: 7.0501x; 1.0626x over previous
import functools

import jax
import jax.numpy as jnp
from jax import lax
from jax.experimental import pallas as pl
from jax.experimental.pallas import tpu as pltpu
from jax.experimental.pallas import tpu_sc as plsc

F32 = jnp.float32
I32 = jnp.int32

_NC = 2
_NS = 16
_LANES = 16
_CHUNK = 128


def _tca1_body(x_ref, wn_ref, bn_ref, alr_ref, nodes_ref, elr_ref, mel_ref):
    nodes = jnp.dot(x_ref[...], wn_ref[...], preferred_element_type=F32)
    nodes = nodes + bn_ref[...]
    nodes_ref[...] = nodes
    elr = jnp.dot(nodes, alr_ref[...], preferred_element_type=F32)
    elr_ref[...] = elr
    mel_ref[...] = jnp.max(elr, axis=0, keepdims=True)


def _tca2_body(ea_ref, aee_ref, bee_ref, ee_ref, mee_ref, *, n_real, blk):
    i = pl.program_id(0)
    ee = jnp.dot(ea_ref[...], aee_ref[...], preferred_element_type=F32)
    ee = ee + bee_ref[...]
    rows = i * blk + lax.broadcasted_iota(I32, ee.shape, 0)
    ee = jnp.where(rows < n_real, ee, -1e30)
    ee_ref[...] = ee
    m = jnp.max(ee, axis=0, keepdims=True)

    @pl.when(i == 0)
    def _():
        mee_ref[...] = m

    @pl.when(i > 0)
    def _():
        mee_ref[...] = jnp.maximum(mee_ref[...], m)


def _tcb_body(sn_ref, sx_ref, f1_ref, wef_ref, biase_ref, convb_ref,
              gamma_ref, beta_ref, hw_ref, hb_ref, hout_ref, pooled_ref,
              *, num_heads, edge_dim):
    hid = f1_ref.shape[1]
    outs = []
    for h in range(num_heads):
        sub = hid * (h % 2)
        sn_h = sn_ref[h // 2][:, sub:sub + hid]
        se_h = sx_ref[h // 2][:, sub:sub + edge_dim]
        esum = sx_ref[h // 2][:, sub + edge_dim:sub + edge_dim + 1]
        r = jnp.where(esum > 0, 1.0 / esum, 0.0)
        s = (esum > 0).astype(F32)
        rst = jnp.dot(sn_h, f1_ref[h], preferred_element_type=F32)
        rst = rst + jnp.dot(se_h, wef_ref[h], preferred_element_type=F32)
        rst = rst * r
        rst = rst + s * biase_ref[pl.ds(h, 1), :]
        rst = rst + convb_ref[pl.ds(h, 1), :]
        rst = rst * jax.nn.sigmoid(rst)
        outs.append(rst)
    n = outs[0].shape[0]
    cnt = float(n * num_heads)
    mu = sum(jnp.sum(o, axis=0, keepdims=True) for o in outs) / cnt
    var = sum(jnp.sum((o - mu) ** 2, axis=0, keepdims=True)
              for o in outs) / cnt
    scale = gamma_ref[...] / jnp.sqrt(var + 1e-5)
    normed = [(o - mu) * scale + beta_ref[...] for o in outs]
    y = jnp.concatenate(normed, axis=1)
    hout = jnp.dot(y, hw_ref[...], preferred_element_type=F32) + hb_ref[...]
    hout_ref[...] = hout
    pooled_ref[...] = jnp.sum(hout, axis=0, keepdims=True) / float(n)


def _sc_main_body(nodes_hbm, eler_hbm, pk_hbm, m_hbm, z128_hbm,
                  sn_hbm, sx_hbm,
                  mv0, mv1, srcb0, dstb0, pkv0, elrows0, errows0, rows0, sem0,
                  srcb1, dstb1, pkv1, elrows1, errows1, rows1, sem1, scl,
                  acc,
                  *, n_pad, e_pad, num_heads, node_dim, edge_dim):
    cid = lax.axis_index("c")
    sid = lax.axis_index("s")
    epw = e_pad // _NS
    nchunk = epw // _CHUNK
    npt = n_pad // _NS

    pltpu.sync_copy(m_hbm.at[cid * 2], mv0)
    pltpu.sync_copy(m_hbm.at[cid * 2 + 1], mv1)

    iot = lax.iota(I32, _LANES)
    ngroups = _CHUNK // _LANES
    rsl = pl.ds(sid * npt, npt)
    sets = ((srcb0, dstb0, pkv0, elrows0, errows0, rows0, sem0),
            (srcb1, dstb1, pkv1, elrows1, errows1, rows1, sem1))

    def zero_acc():
        pltpu.sync_copy(z128_hbm, scl)
        for q in range(npt // _CHUNK):
            qsl = pl.ds(sid * npt + q * _CHUNK, _CHUNK)
            pltpu.sync_copy(scl, acc.at[qsl])

    def issue_pk(b, k):
        srcb, dstb, pkv, elrows, errows, rows, sem = sets[b]
        base = sid * epw + k * _CHUNK
        pltpu.async_copy(pk_hbm.at[pl.ds(base, _CHUNK), :], pkv, sem)

    def wait_pk(b):
        srcb, dstb, pkv, elrows, errows, rows, sem = sets[b]
        pltpu.make_async_copy(pk_hbm.at[pl.ds(0, _CHUNK), :], pkv, sem).wait()

    def extract(b):
        srcb, dstb, pkv, elrows, errows, rows, sem = sets[b]
        for g in range(ngroups):
            eloc = g * _LANES + iot
            sv = plsc.load_gather(pkv, [eloc, jnp.zeros((_LANES,), I32)])
            dv = plsc.load_gather(pkv, [eloc, jnp.full((_LANES,), 1, I32)])
            srcb[pl.ds(g * _LANES, _LANES)] = plsc.bitcast(sv, I32)
            dstb[pl.ds(g * _LANES, _LANES)] = plsc.bitcast(dv, I32)

    def issue_ind(b, with_rows):
        srcb, dstb, pkv, elrows, errows, rows, sem = sets[b]
        pltpu.async_copy(eler_hbm.at[srcb], elrows, sem)
        pltpu.async_copy(eler_hbm.at[dstb], errows, sem)
        if with_rows:
            pltpu.async_copy(nodes_hbm.at[srcb], rows, sem)

    def wait_ind(b, with_rows):
        srcb, dstb, pkv, elrows, errows, rows, sem = sets[b]
        pltpu.make_async_copy(eler_hbm.at[srcb], elrows, sem).wait()
        pltpu.make_async_copy(eler_hbm.at[dstb], errows, sem).wait()
        if with_rows:
            pltpu.make_async_copy(nodes_hbm.at[srcb], rows, sem).wait()

    def eexp(b, hl, g):
        srcb, dstb, pkv, elrows, errows, rows, sem = sets[b]
        h16 = jnp.full((_LANES,), cid * 2 + hl, I32)
        mvec = (mv0, mv1)[hl][...]
        eloc = g * _LANES + iot
        elg = plsc.load_gather(elrows, [eloc, h16])
        erg = plsc.load_gather(errows, [eloc, h16 + num_heads])
        eeg = plsc.load_gather(pkv, [eloc, h16 + 2])
        z = elg + erg + eeg
        zl = jnp.where(z > 0.0, z, 0.2 * z)
        cg = jnp.maximum(erg + mvec, 0.0)
        return eloc, jnp.exp(zl - cg)

    def compute_a(b):
        srcb, dstb, pkv, elrows, errows, rows, sem = sets[b]
        for hl in range(2):
            cbase = node_dim * hl

            def grp_a(g, carry2):
                eloc, exv = eexp(b, hl, g)
                for j in range(node_dim):
                    dj16 = jnp.full((_LANES,), j, I32)
                    v = plsc.load_gather(rows, [eloc, dj16])
                    plsc.store_scatter(scl, [eloc, dj16 + cbase], v * exv)
                return carry2

            lax.fori_loop(0, ngroups, grp_a, 0)
        pltpu.sync_copy(scl, acc.at[dstb], add=True)

    def compute_b(b):
        srcb, dstb, pkv, elrows, errows, rows, sem = sets[b]
        for hl in range(2):
            cbase = node_dim * hl

            def grp_b(g, carry2):
                eloc, exv = eexp(b, hl, g)
                plsc.store_scatter(
                    scl, [eloc, jnp.full((_LANES,), cbase + edge_dim, I32)],
                    exv)
                for j in range(edge_dim):
                    j16 = jnp.full((_LANES,), j + 2 + num_heads, I32)
                    v = plsc.load_gather(pkv, [eloc, j16])
                    plsc.store_scatter(
                        scl, [eloc, jnp.full((_LANES,), j + cbase, I32)],
                        v * exv)
                return carry2

            lax.fori_loop(0, ngroups, grp_b, 0)
        pltpu.sync_copy(scl, acc.at[dstb], add=True)

    def run_phase(compute, with_rows):
        zero_acc()
        plsc.subcore_barrier()
        issue_pk(0, 0)
        issue_pk(1, 1)
        wait_pk(0)
        extract(0)
        issue_ind(0, with_rows)

        def pair(g, carry):
            a = 2 * g
            wait_pk(1)
            extract(1)
            issue_ind(1, with_rows)
            wait_ind(0, with_rows)
            compute(0)

            @pl.when(a + 2 < nchunk)
            def _():
                issue_pk(0, a + 2)

            @pl.when(a + 2 < nchunk)
            def _():
                wait_pk(0)
                extract(0)
                issue_ind(0, with_rows)

            wait_ind(1, with_rows)
            compute(1)

            @pl.when(a + 3 < nchunk)
            def _():
                issue_pk(1, a + 3)

            return carry

        lax.fori_loop(0, nchunk // 2, pair, 0)
        plsc.subcore_barrier()

    run_phase(compute_a, True)
    pltpu.sync_copy(acc.at[rsl], sn_hbm.at[cid, rsl, :])
    plsc.subcore_barrier()

    run_phase(compute_b, False)
    plsc.subcore_barrier()
    pltpu.sync_copy(acc.at[rsl], sx_hbm.at[cid, rsl, :])


def _sc_gather_body(hout_hbm, idx_hbm, out_hbm, idxv, rowsv, sem, *, b_per_w):
    wid = lax.axis_index("s") * _NC + lax.axis_index("c")
    base = wid * b_per_w
    pltpu.sync_copy(idx_hbm.at[pl.ds(base, b_per_w)], idxv)
    pltpu.async_copy(hout_hbm.at[idxv], rowsv, sem).wait()
    pltpu.sync_copy(rowsv, out_hbm.at[pl.ds(base, b_per_w)])


def kernel(x, edge_attr, edge_index, nodes_idx, Wn, b_node, We, b_edge,
           fc_W, fce_W, attn_l, attn_r, attn_e, conv_b, gamma, beta,
           head_W, head_b):
    n_nodes, node_dim_in = x.shape
    n_edges, edge_dim = edge_attr.shape
    hid = Wn.shape[1]
    num_heads = attn_l.shape[1]
    n_idx = nodes_idx.shape[0]

    fc1r = fc_W[-1].reshape(hid, num_heads, hid)
    fce1r = fce_W[-1].reshape(hid, num_heads, hid)
    al = jnp.einsum("khd,hd->kh", fc1r, attn_l[-1])
    ar = jnp.einsum("khd,hd->kh", fc1r, attn_r[-1])
    ae = jnp.einsum("khd,hd->kh", fce1r, attn_e[-1])
    alr = jnp.concatenate([al, ar], axis=1)
    aee = We @ ae
    bee = (b_edge @ ae).reshape(1, num_heads)
    f1 = jnp.transpose(fc1r, (1, 0, 2))
    fce1h = jnp.transpose(fce1r, (1, 0, 2))
    wef = jnp.einsum("ke,hed->hkd", We, fce1h)
    bias_e = jnp.einsum("e,hed->hd", b_edge, fce1h)

    n_pad = -(-n_nodes // (_NS * _CHUNK)) * (_NS * _CHUNK)
    e_pad = -(-n_edges // (2 * _NS * _CHUNK)) * (2 * _NS * _CHUNK)
    pad = e_pad - n_edges
    src = jnp.concatenate([edge_index[0], jnp.zeros((pad,), I32)])
    dst = jnp.concatenate([edge_index[1], jnp.zeros((pad,), I32)])
    eap = jnp.concatenate([edge_attr, jnp.zeros((pad, edge_dim), F32)], axis=0)

    nodes, elr, melr = pl.pallas_call(
        _tca1_body,
        out_shape=[
            jax.ShapeDtypeStruct((n_nodes, hid), F32),
            jax.ShapeDtypeStruct((n_nodes, 2 * num_heads), F32),
            jax.ShapeDtypeStruct((1, 2 * num_heads), F32),
        ],
    )(x, Wn, b_node.reshape(1, hid), alr)
    zrow = jnp.zeros((n_pad - n_nodes, num_heads), F32)
    el = jnp.concatenate([elr[:, :num_heads], zrow], axis=0)
    er = jnp.concatenate([elr[:, num_heads:], zrow], axis=0)

    gblk = e_pad // _NS
    ee, mee = pl.pallas_call(
        functools.partial(_tca2_body, n_real=n_edges, blk=gblk),
        grid=(_NS,),
        in_specs=[
            pl.BlockSpec((gblk, edge_dim), lambda i: (i, 0)),
            pl.BlockSpec((edge_dim, num_heads), lambda i: (0, 0)),
            pl.BlockSpec((1, num_heads), lambda i: (0, 0)),
        ],
        out_specs=[
            pl.BlockSpec((gblk, num_heads), lambda i: (i, 0)),
            pl.BlockSpec((1, num_heads), lambda i: (0, 0)),
        ],
        out_shape=[
            jax.ShapeDtypeStruct((e_pad, num_heads), F32),
            jax.ShapeDtypeStruct((1, num_heads), F32),
        ],
    )(eap, aee, bee)

    mstab = jnp.tile((melr[0, :num_heads] + mee[0]).reshape(num_heads, 1),
                     (1, _LANES))
    z128 = jnp.zeros((_CHUNK, 2 * hid), F32)

    eler = jnp.concatenate(
        [el, er, jnp.zeros((n_pad, _LANES - 2 * num_heads), F32)],
        axis=1)
    pkw = 2 + num_heads + edge_dim + 2
    pk = jnp.concatenate([
        lax.bitcast_convert_type(src, F32).reshape(e_pad, 1),
        lax.bitcast_convert_type(dst, F32).reshape(e_pad, 1),
        ee,
        eap,
        jnp.zeros((e_pad, 2), F32),
    ], axis=1)
    mesh = plsc.VectorSubcoreMesh(core_axis_name="c", subcore_axis_name="s")
    sc_main = functools.partial(
        pl.kernel,
        functools.partial(_sc_main_body, n_pad=n_pad, e_pad=e_pad,
                          num_heads=num_heads, node_dim=hid,
                          edge_dim=edge_dim),
        out_type=[
            jax.ShapeDtypeStruct((_NC, n_pad, 2 * hid), F32),
            jax.ShapeDtypeStruct((_NC, n_pad, 2 * hid), F32),
        ],
        mesh=mesh,
        scratch_types=(
            [pltpu.VMEM((_LANES,), F32)] * 2 +
            [pltpu.VMEM((_CHUNK,), I32),
             pltpu.VMEM((_CHUNK,), I32),
             pltpu.VMEM((_CHUNK, pkw), F32),
             pltpu.VMEM((_CHUNK, _LANES), F32),
             pltpu.VMEM((_CHUNK, _LANES), F32),
             pltpu.VMEM((_CHUNK, hid), F32),
             pltpu.SemaphoreType.DMA] * 2 +
            [pltpu.VMEM((_CHUNK, 2 * hid), F32),
             pltpu.VMEM_SHARED((n_pad, 2 * hid), F32)]
        ),
        compiler_params=pltpu.CompilerParams(needs_layout_passes=False, use_tc_tiling_on_sc=False),
    )()
    sn, sx = sc_main(nodes, eler, pk, mstab, z128)
    sn = sn[:, :n_nodes, :]
    sx = sx[:, :n_nodes, :]

    hout, pooled = pl.pallas_call(
        functools.partial(_tcb_body, num_heads=num_heads, edge_dim=edge_dim),
        out_shape=[
            jax.ShapeDtypeStruct((n_nodes, hid), F32),
            jax.ShapeDtypeStruct((1, hid), F32),
        ],
        compiler_params=pltpu.CompilerParams(
            vmem_limit_bytes=100 * 1024 * 1024),
    )(sn, sx, f1, wef, bias_e, conv_b[-1].reshape(num_heads, hid),
      gamma[-1].reshape(1, hid), beta[-1].reshape(1, hid), head_W,
      head_b.reshape(1, hid))

    b_per_w = n_idx // (_NC * _NS)
    sc_gather = functools.partial(
        pl.kernel,
        functools.partial(_sc_gather_body, b_per_w=b_per_w),
        out_type=jax.ShapeDtypeStruct((n_idx, hid), F32),
        mesh=mesh,
        scratch_types=[
            pltpu.VMEM((b_per_w,), I32),
            pltpu.VMEM((b_per_w, hid), F32),
            pltpu.SemaphoreType.DMA,
        ],
        compiler_params=pltpu.CompilerParams(needs_layout_passes=False, use_tc_tiling_on_sc=False),
    )()
    gathered = sc_gather(hout, nodes_idx)

    return pooled, gathered
